# baseline (speedup 1.0000x reference)
import functools

import jax
import jax.numpy as jnp
from jax import lax
from jax.experimental import pallas as pl
from jax.experimental.pallas import tpu as pltpu
from jax.experimental.pallas import tpu_sc as plsc


def _time_tc(t, time_w, time_b):
    E = t.shape[0]
    DT = time_w.shape[0]
    B = 1280
    assert E % B == 0
    grid = E // B

    def body(t_ref, w_ref, b_ref, o_ref):
        o_ref[...] = jnp.cos(t_ref[...] * w_ref[...] + b_ref[...])

    return pl.pallas_call(
        body,
        grid=(grid,),
        in_specs=[
            pl.BlockSpec((B, 1), lambda i: (i, 0)),
            pl.BlockSpec((1, DT), lambda i: (0, 0)),
            pl.BlockSpec((1, DT), lambda i: (0, 0)),
        ],
        out_specs=pl.BlockSpec((B, DT), lambda i: (i, 0)),
        out_shape=jax.ShapeDtypeStruct((E, DT), jnp.float32),
    )(t.reshape(E, 1), time_w.reshape(1, DT), time_b.reshape(1, DT))


def _sc_assemble(table, fidx2, tidx2, edge3, time3, out_w):
    CH, C = fidx2.shape
    N, D = table.shape
    DE = edge3.shape[-1]
    DT = time3.shape[-1]
    info = plsc.get_sparse_core_info()
    NC = info.num_cores
    NW = NC * info.num_subcores
    n_iter = -(-CH // NW)
    mesh = plsc.VectorSubcoreMesh(core_axis_name="c", subcore_axis_name="s")

    S = 3

    @functools.partial(
        pl.kernel,
        mesh=mesh,
        compiler_params=pltpu.CompilerParams(use_tc_tiling_on_sc=False),
        out_type=jax.ShapeDtypeStruct((CH, C, out_w), jnp.float32),
        scratch_types=[
            pltpu.VMEM((S, C), jnp.int32),
            pltpu.VMEM((S, C), jnp.int32),
            pltpu.VMEM((S, C, D), jnp.float32),
            pltpu.VMEM((S, C, D), jnp.float32),
            pltpu.SemaphoreType.DMA((S,)),
            pltpu.SemaphoreType.DMA((S,)),
            pltpu.SemaphoreType.DMA((S,)),
            pltpu.SemaphoreType.DMA((S,)),
        ],
    )
    def k(table_h, fidx_h, tidx_h, edge_h, time_h, out_h,
          fiv, tiv, fr, tr, sem_a, sem_b, sem_c, sem_d):
        wid = lax.axis_index("s") * NC + lax.axis_index("c")

        def stage_a(g):
            ch = g * NW + wid
            s = g % S
            pltpu.async_copy(fidx_h.at[ch], fiv.at[s], sem_a.at[s])
            pltpu.async_copy(tidx_h.at[ch], tiv.at[s], sem_a.at[s])
            pltpu.async_copy(edge_h.at[ch], out_h.at[ch, :, pl.ds(D, DE)], sem_d.at[s])
            pltpu.async_copy(time_h.at[ch], out_h.at[ch, :, pl.ds(2 * D + DE, DT)], sem_d.at[s])

        def stage_b(g):
            ch = g * NW + wid
            s = g % S
            pltpu.make_async_copy(fidx_h.at[ch], fiv.at[s], sem_a.at[s]).wait()
            pltpu.make_async_copy(tidx_h.at[ch], tiv.at[s], sem_a.at[s]).wait()
            pltpu.async_copy(table_h.at[fiv.at[s]], fr.at[s], sem_b.at[s])
            pltpu.async_copy(table_h.at[tiv.at[s]], tr.at[s], sem_b.at[s])

        def stage_c(g):
            ch = g * NW + wid
            s = g % S
            pltpu.make_async_copy(table_h.at[fiv.at[s]], fr.at[s], sem_b.at[s]).wait()
            pltpu.make_async_copy(table_h.at[tiv.at[s]], tr.at[s], sem_b.at[s]).wait()
            pltpu.async_copy(fr.at[s], out_h.at[ch, :, pl.ds(0, D)], sem_c.at[s])
            pltpu.async_copy(tr.at[s], out_h.at[ch, :, pl.ds(D + DE, D)], sem_c.at[s])

        def stage_w(g):
            ch = g * NW + wid
            s = g % S
            pltpu.make_async_copy(fr.at[s], out_h.at[ch, :, pl.ds(0, D)], sem_c.at[s]).wait()
            pltpu.make_async_copy(tr.at[s], out_h.at[ch, :, pl.ds(D + DE, D)], sem_c.at[s]).wait()
            pltpu.make_async_copy(edge_h.at[ch], out_h.at[ch, :, pl.ds(D, DE)], sem_d.at[s]).wait()
            pltpu.make_async_copy(time_h.at[ch], out_h.at[ch, :, pl.ds(2 * D + DE, DT)], sem_d.at[s]).wait()

        def live(g):
            return jnp.logical_and(g >= 0, (g * NW + wid) < CH)

        def body(g, carry):
            @pl.when(live(g - S))
            def _():
                stage_w(g - S)

            @pl.when(live(g))
            def _():
                stage_a(g)

            @pl.when(live(g - 1))
            def _():
                stage_b(g - 1)

            @pl.when(live(g - 2))
            def _():
                stage_c(g - 2)

            return carry

        lax.fori_loop(0, n_iter + S, body, 0)

    return k(table, fidx2, tidx2, edge3, time3)


def kernel(update_node_emb, edge_emb, from_idx, to_idx, t, time_w, time_b):
    N, D = update_node_emb.shape
    E, DE = edge_emb.shape
    DT = time_w.shape[0]
    out_w = D + DE + D + DT
    C = 128
    assert E % C == 0
    CH = E // C

    time_emb = _time_tc(t, time_w, time_b)
    fidx2 = from_idx.astype(jnp.int32).reshape(CH, C)
    tidx2 = to_idx.astype(jnp.int32).reshape(CH, C)
    edge3 = edge_emb.reshape(CH, C, DE)
    time3 = time_emb.reshape(CH, C, DT)
    out3 = _sc_assemble(update_node_emb, fidx2, tidx2, edge3, time3, out_w)
    return out3.reshape(E, out_w)


# speedup vs baseline: 3.7165x; 3.7165x over previous
import functools

import jax
import jax.numpy as jnp
from jax import lax
from jax.experimental import pallas as pl
from jax.experimental.pallas import tpu as pltpu
from jax.experimental.pallas import tpu_sc as plsc


def _time_tc(t, time_w, time_b):
    E = t.shape[0]
    DT = time_w.shape[0]
    B = 1280
    assert E % B == 0
    grid = E // B

    def body(t_ref, w_ref, b_ref, o_ref):
        o_ref[...] = jnp.cos(t_ref[...] * w_ref[...] + b_ref[...])

    return pl.pallas_call(
        body,
        grid=(grid,),
        in_specs=[
            pl.BlockSpec((B, 1), lambda i: (i, 0)),
            pl.BlockSpec((1, DT), lambda i: (0, 0)),
            pl.BlockSpec((1, DT), lambda i: (0, 0)),
        ],
        out_specs=pl.BlockSpec((B, DT), lambda i: (i, 0)),
        out_shape=jax.ShapeDtypeStruct((E, DT), jnp.float32),
    )(t.reshape(E, 1), time_w.reshape(1, DT), time_b.reshape(1, DT))


def _sc_assemble(table, fidx2, tidx2, edge3, time3, out_w):
    CH, C = fidx2.shape
    N, D = table.shape
    DE = edge3.shape[-1]
    DT = time3.shape[-1]
    info = plsc.get_sparse_core_info()
    NC = info.num_cores
    NW = NC * info.num_subcores
    n_iter = -(-CH // NW)
    mesh = plsc.VectorSubcoreMesh(core_axis_name="c", subcore_axis_name="s")

    S = 2

    @functools.partial(
        pl.kernel,
        mesh=mesh,
        compiler_params=pltpu.CompilerParams(use_tc_tiling_on_sc=False),
        out_type=jax.ShapeDtypeStruct((CH, C, out_w), jnp.float32),
        scratch_types=[
            pltpu.VMEM((S, C), jnp.int32),
            pltpu.VMEM((S, C), jnp.int32),
            pltpu.VMEM((S, C, D), jnp.float32),
            pltpu.VMEM((S, C, D), jnp.float32),
            pltpu.VMEM((S, C, DE), jnp.float32),
            pltpu.VMEM((S, C, DT), jnp.float32),
            pltpu.SemaphoreType.DMA((S,)),
            pltpu.SemaphoreType.DMA((S,)),
            pltpu.SemaphoreType.DMA((S,)),
            pltpu.SemaphoreType.DMA((S,)),
        ],
    )
    def k(table_h, fidx_h, tidx_h, edge_h, time_h, out_h,
          fiv, tiv, fr, tr, eb, tb, sem_i, sem_g, sem_l, sem_w):
        wid = lax.axis_index("s") * NC + lax.axis_index("c")

        def stage_in(g):
            ch = g * NW + wid
            s = g % S
            pltpu.async_copy(fidx_h.at[ch], fiv.at[s], sem_i.at[s])
            pltpu.async_copy(tidx_h.at[ch], tiv.at[s], sem_i.at[s])
            pltpu.async_copy(edge_h.at[ch], eb.at[s], sem_l.at[s])
            pltpu.async_copy(time_h.at[ch], tb.at[s], sem_l.at[s])

        def stage_gather(g):
            ch = g * NW + wid
            s = g % S
            pltpu.make_async_copy(fidx_h.at[ch], fiv.at[s], sem_i.at[s]).wait()
            pltpu.make_async_copy(tidx_h.at[ch], tiv.at[s], sem_i.at[s]).wait()
            pltpu.async_copy(table_h.at[fiv.at[s]], fr.at[s], sem_g.at[s])
            pltpu.async_copy(table_h.at[tiv.at[s]], tr.at[s], sem_g.at[s])

        def stage_write_et(g):
            ch = g * NW + wid
            s = g % S
            pltpu.make_async_copy(edge_h.at[ch], eb.at[s], sem_l.at[s]).wait()
            pltpu.make_async_copy(time_h.at[ch], tb.at[s], sem_l.at[s]).wait()
            pltpu.async_copy(eb.at[s], out_h.at[ch, :, pl.ds(D, DE)], sem_w.at[s])
            pltpu.async_copy(tb.at[s], out_h.at[ch, :, pl.ds(2 * D + DE, DT)], sem_w.at[s])

        def stage_write_ft(g):
            ch = g * NW + wid
            s = g % S
            pltpu.make_async_copy(table_h.at[fiv.at[s]], fr.at[s], sem_g.at[s]).wait()
            pltpu.make_async_copy(table_h.at[tiv.at[s]], tr.at[s], sem_g.at[s]).wait()
            pltpu.async_copy(fr.at[s], out_h.at[ch, :, pl.ds(0, D)], sem_w.at[s])
            pltpu.async_copy(tr.at[s], out_h.at[ch, :, pl.ds(D + DE, D)], sem_w.at[s])

        def stage_drain(g):
            ch = g * NW + wid
            s = g % S
            pltpu.make_async_copy(eb.at[s], out_h.at[ch, :, pl.ds(D, DE)], sem_w.at[s]).wait()
            pltpu.make_async_copy(tb.at[s], out_h.at[ch, :, pl.ds(2 * D + DE, DT)], sem_w.at[s]).wait()
            pltpu.make_async_copy(fr.at[s], out_h.at[ch, :, pl.ds(0, D)], sem_w.at[s]).wait()
            pltpu.make_async_copy(tr.at[s], out_h.at[ch, :, pl.ds(D + DE, D)], sem_w.at[s]).wait()

        def live(g):
            return jnp.logical_and(g >= 0, (g * NW + wid) < CH)

        def body(g, carry):
            @pl.when(live(g - 2))
            def _():
                stage_drain(g - 2)

            @pl.when(live(g))
            def _():
                stage_in(g)

            @pl.when(live(g - 1))
            def _():
                stage_write_et(g - 1)
                stage_write_ft(g - 1)

            @pl.when(live(g))
            def _():
                stage_gather(g)

            return carry

        lax.fori_loop(0, n_iter + S, body, 0)

    return k(table, fidx2, tidx2, edge3, time3)


def kernel(update_node_emb, edge_emb, from_idx, to_idx, t, time_w, time_b):
    N, D = update_node_emb.shape
    E, DE = edge_emb.shape
    DT = time_w.shape[0]
    out_w = D + DE + D + DT
    C = 128
    assert E % C == 0
    CH = E // C

    time_emb = _time_tc(t, time_w, time_b)
    fidx2 = from_idx.astype(jnp.int32).reshape(CH, C)
    tidx2 = to_idx.astype(jnp.int32).reshape(CH, C)
    edge3 = edge_emb.reshape(CH, C, DE)
    time3 = time_emb.reshape(CH, C, DT)
    out3 = _sc_assemble(update_node_emb, fidx2, tidx2, edge3, time3, out_w)
    return out3.reshape(E, out_w)


# speedup vs baseline: 4.8152x; 1.2956x over previous
import functools

import jax
import jax.numpy as jnp
from jax import lax
from jax.experimental import pallas as pl
from jax.experimental.pallas import tpu as pltpu
from jax.experimental.pallas import tpu_sc as plsc


_INV2PI = 0.15915493667125702
_TWOPI = 6.283185307179586
_CC = (0.999999443678766, -0.49999558165578417, 0.04166103279005172,
       -0.001386274731578642, 2.425319249599542e-05, -2.2193949944101022e-07)


def _time_tc(t, time_w, time_b):
    E = t.shape[0]
    DT = time_w.shape[0]
    BR = 20
    C = 128
    G = E // (BR * C)
    assert E == G * BR * C

    def body(t_ref, w_ref, b_ref, o_ref):
        x = t_ref[0][:, :, None] * w_ref[...] + b_ref[...]
        r = x - jnp.floor(x * _INV2PI + 0.5) * _TWOPI
        u = r * r
        p = jnp.float32(_CC[5])
        for c in (_CC[4], _CC[3], _CC[2], _CC[1], _CC[0]):
            p = p * u + c
        o_ref[...] = p.reshape(1, BR * C, DT)

    out = pl.pallas_call(
        body,
        grid=(G,),
        in_specs=[
            pl.BlockSpec((1, BR, C), lambda i: (i, 0, 0)),
            pl.BlockSpec((1, 1, DT), lambda i: (0, 0, 0)),
            pl.BlockSpec((1, 1, DT), lambda i: (0, 0, 0)),
        ],
        out_specs=pl.BlockSpec((1, BR * C, DT), lambda i: (i, 0, 0)),
        out_shape=jax.ShapeDtypeStruct((G, BR * C, DT), jnp.float32),
    )(t.reshape(G, BR, C), time_w.reshape(1, 1, DT), time_b.reshape(1, 1, DT))
    return out.reshape(E // C, C, DT)


def _sc_assemble(table, fidx2, tidx2, edge3, time3, out_w):
    CH, C = fidx2.shape
    N, D = table.shape
    DE = edge3.shape[-1]
    DT = time3.shape[-1]
    info = plsc.get_sparse_core_info()
    NC = info.num_cores
    NW = NC * info.num_subcores
    n_iter = -(-CH // NW)
    mesh = plsc.VectorSubcoreMesh(core_axis_name="c", subcore_axis_name="s")

    S = 2

    @functools.partial(
        pl.kernel,
        mesh=mesh,
        compiler_params=pltpu.CompilerParams(use_tc_tiling_on_sc=False),
        out_type=jax.ShapeDtypeStruct((CH, C, out_w), jnp.float32),
        scratch_types=[
            pltpu.VMEM((S, C), jnp.int32),
            pltpu.VMEM((S, C), jnp.int32),
            pltpu.VMEM((S, C, D), jnp.float32),
            pltpu.VMEM((S, C, D), jnp.float32),
            pltpu.VMEM((S, C, DE), jnp.float32),
            pltpu.VMEM((S, C, DT), jnp.float32),
            pltpu.SemaphoreType.DMA((S,)),
            pltpu.SemaphoreType.DMA((S,)),
            pltpu.SemaphoreType.DMA((S,)),
            pltpu.SemaphoreType.DMA((S,)),
        ],
    )
    def k(table_h, fidx_h, tidx_h, edge_h, time_h, out_h,
          fiv, tiv, fr, tr, eb, tb, sem_i, sem_g, sem_l, sem_w):
        wid = lax.axis_index("s") * NC + lax.axis_index("c")

        def stage_in(g):
            ch = g * NW + wid
            s = g % S
            pltpu.async_copy(fidx_h.at[ch], fiv.at[s], sem_i.at[s])
            pltpu.async_copy(tidx_h.at[ch], tiv.at[s], sem_i.at[s])
            pltpu.async_copy(edge_h.at[ch], eb.at[s], sem_l.at[s])
            pltpu.async_copy(time_h.at[ch], tb.at[s], sem_l.at[s])

        def stage_gather(g):
            ch = g * NW + wid
            s = g % S
            pltpu.make_async_copy(fidx_h.at[ch], fiv.at[s], sem_i.at[s]).wait()
            pltpu.make_async_copy(tidx_h.at[ch], tiv.at[s], sem_i.at[s]).wait()
            pltpu.async_copy(table_h.at[fiv.at[s]], fr.at[s], sem_g.at[s])
            pltpu.async_copy(table_h.at[tiv.at[s]], tr.at[s], sem_g.at[s])

        def stage_write_et(g):
            ch = g * NW + wid
            s = g % S
            pltpu.make_async_copy(edge_h.at[ch], eb.at[s], sem_l.at[s]).wait()
            pltpu.make_async_copy(time_h.at[ch], tb.at[s], sem_l.at[s]).wait()
            pltpu.async_copy(eb.at[s], out_h.at[ch, :, pl.ds(D, DE)], sem_w.at[s])
            pltpu.async_copy(tb.at[s], out_h.at[ch, :, pl.ds(2 * D + DE, DT)], sem_w.at[s])

        def stage_write_ft(g):
            ch = g * NW + wid
            s = g % S
            pltpu.make_async_copy(table_h.at[fiv.at[s]], fr.at[s], sem_g.at[s]).wait()
            pltpu.make_async_copy(table_h.at[tiv.at[s]], tr.at[s], sem_g.at[s]).wait()
            pltpu.async_copy(fr.at[s], out_h.at[ch, :, pl.ds(0, D)], sem_w.at[s])
            pltpu.async_copy(tr.at[s], out_h.at[ch, :, pl.ds(D + DE, D)], sem_w.at[s])

        def stage_drain(g):
            ch = g * NW + wid
            s = g % S
            pltpu.make_async_copy(eb.at[s], out_h.at[ch, :, pl.ds(D, DE)], sem_w.at[s]).wait()
            pltpu.make_async_copy(tb.at[s], out_h.at[ch, :, pl.ds(2 * D + DE, DT)], sem_w.at[s]).wait()
            pltpu.make_async_copy(fr.at[s], out_h.at[ch, :, pl.ds(0, D)], sem_w.at[s]).wait()
            pltpu.make_async_copy(tr.at[s], out_h.at[ch, :, pl.ds(D + DE, D)], sem_w.at[s]).wait()

        def live(g):
            return jnp.logical_and(g >= 0, (g * NW + wid) < CH)

        def body(g, carry):
            @pl.when(live(g - 2))
            def _():
                stage_drain(g - 2)

            @pl.when(live(g))
            def _():
                stage_in(g)

            @pl.when(live(g - 1))
            def _():
                stage_write_et(g - 1)
                stage_write_ft(g - 1)

            @pl.when(live(g))
            def _():
                stage_gather(g)

            return carry

        lax.fori_loop(0, n_iter + S, body, 0)

    return k(table, fidx2, tidx2, edge3, time3)


def kernel(update_node_emb, edge_emb, from_idx, to_idx, t, time_w, time_b):
    N, D = update_node_emb.shape
    E, DE = edge_emb.shape
    DT = time_w.shape[0]
    out_w = D + DE + D + DT
    C = 128
    assert E % C == 0
    CH = E // C

    time3 = _time_tc(t, time_w, time_b)
    fidx2 = from_idx.astype(jnp.int32).reshape(CH, C)
    tidx2 = to_idx.astype(jnp.int32).reshape(CH, C)
    edge3 = edge_emb.reshape(CH, C, DE)
    out3 = _sc_assemble(update_node_emb, fidx2, tidx2, edge3, time3, out_w)
    return out3.reshape(E, out_w)


# speedup vs baseline: 4.8167x; 1.0003x over previous
import functools

import jax
import jax.numpy as jnp
from jax import lax
from jax.experimental import pallas as pl
from jax.experimental.pallas import tpu as pltpu
from jax.experimental.pallas import tpu_sc as plsc


_INV2PI = 0.15915493667125702
_TWOPI = 6.283185307179586
_CC = (0.999999443678766, -0.49999558165578417, 0.04166103279005172,
       -0.001386274731578642, 2.425319249599542e-05, -2.2193949944101022e-07)


def _time_tc(t, time_w, time_b):
    E = t.shape[0]
    DT = time_w.shape[0]
    BR = 20
    C = 128
    G = E // (BR * C)
    assert E == G * BR * C

    def body(t_ref, w_ref, b_ref, o_ref):
        x = t_ref[0][:, :, None] * w_ref[...] + b_ref[...]
        r = x - jnp.floor(x * _INV2PI + 0.5) * _TWOPI
        u = r * r
        p = jnp.float32(_CC[5])
        for c in (_CC[4], _CC[3], _CC[2], _CC[1], _CC[0]):
            p = p * u + c
        o_ref[...] = p.reshape(BR * C, DT)

    return pl.pallas_call(
        body,
        grid=(G,),
        in_specs=[
            pl.BlockSpec((1, BR, C), lambda i: (i, 0, 0)),
            pl.BlockSpec((1, 1, DT), lambda i: (0, 0, 0)),
            pl.BlockSpec((1, 1, DT), lambda i: (0, 0, 0)),
        ],
        out_specs=pl.BlockSpec((BR * C, DT), lambda i: (i, 0)),
        out_shape=jax.ShapeDtypeStruct((E, DT), jnp.float32),
    )(t.reshape(G, BR, C), time_w.reshape(1, 1, DT), time_b.reshape(1, 1, DT))


def _sc_assemble(table, fidx, tidx, edge2, time2, out_w):
    E = fidx.shape[0]
    N, D = table.shape
    DE = edge2.shape[-1]
    DT = time2.shape[-1]
    C = 128
    CH = E // C
    info = plsc.get_sparse_core_info()
    NC = info.num_cores
    NW = NC * info.num_subcores
    n_iter = -(-CH // NW)
    mesh = plsc.VectorSubcoreMesh(core_axis_name="c", subcore_axis_name="s")

    S = 2

    @functools.partial(
        pl.kernel,
        mesh=mesh,
        compiler_params=pltpu.CompilerParams(use_tc_tiling_on_sc=False),
        out_type=jax.ShapeDtypeStruct((E, out_w), jnp.float32),
        scratch_types=[
            pltpu.VMEM((S, C), jnp.int32),
            pltpu.VMEM((S, C), jnp.int32),
            pltpu.VMEM((S, C, D), jnp.float32),
            pltpu.VMEM((S, C, D), jnp.float32),
            pltpu.VMEM((S, C, DE), jnp.float32),
            pltpu.VMEM((S, C, DT), jnp.float32),
            pltpu.SemaphoreType.DMA((S,)),
            pltpu.SemaphoreType.DMA((S,)),
            pltpu.SemaphoreType.DMA((S,)),
            pltpu.SemaphoreType.DMA((S,)),
        ],
    )
    def k(table_h, fidx_h, tidx_h, edge_h, time_h, out_h,
          fiv, tiv, fr, tr, eb, tb, sem_i, sem_g, sem_l, sem_w):
        wid = lax.axis_index("s") * NC + lax.axis_index("c")

        def stage_in(g):
            r0 = (g * NW + wid) * C
            s = g % S
            pltpu.async_copy(fidx_h.at[pl.ds(r0, C)], fiv.at[s], sem_i.at[s])
            pltpu.async_copy(tidx_h.at[pl.ds(r0, C)], tiv.at[s], sem_i.at[s])
            pltpu.async_copy(edge_h.at[pl.ds(r0, C)], eb.at[s], sem_l.at[s])
            pltpu.async_copy(time_h.at[pl.ds(r0, C)], tb.at[s], sem_l.at[s])

        def stage_gather(g):
            r0 = (g * NW + wid) * C
            s = g % S
            pltpu.make_async_copy(fidx_h.at[pl.ds(r0, C)], fiv.at[s], sem_i.at[s]).wait()
            pltpu.make_async_copy(tidx_h.at[pl.ds(r0, C)], tiv.at[s], sem_i.at[s]).wait()
            pltpu.async_copy(table_h.at[fiv.at[s]], fr.at[s], sem_g.at[s])
            pltpu.async_copy(table_h.at[tiv.at[s]], tr.at[s], sem_g.at[s])

        def stage_write_et(g):
            r0 = (g * NW + wid) * C
            s = g % S
            pltpu.make_async_copy(edge_h.at[pl.ds(r0, C)], eb.at[s], sem_l.at[s]).wait()
            pltpu.make_async_copy(time_h.at[pl.ds(r0, C)], tb.at[s], sem_l.at[s]).wait()
            pltpu.async_copy(eb.at[s], out_h.at[pl.ds(r0, C), pl.ds(D, DE)], sem_w.at[s])
            pltpu.async_copy(tb.at[s], out_h.at[pl.ds(r0, C), pl.ds(2 * D + DE, DT)], sem_w.at[s])

        def stage_write_ft(g):
            r0 = (g * NW + wid) * C
            s = g % S
            pltpu.make_async_copy(table_h.at[fiv.at[s]], fr.at[s], sem_g.at[s]).wait()
            pltpu.make_async_copy(table_h.at[tiv.at[s]], tr.at[s], sem_g.at[s]).wait()
            pltpu.async_copy(fr.at[s], out_h.at[pl.ds(r0, C), pl.ds(0, D)], sem_w.at[s])
            pltpu.async_copy(tr.at[s], out_h.at[pl.ds(r0, C), pl.ds(D + DE, D)], sem_w.at[s])

        def stage_drain(g):
            r0 = (g * NW + wid) * C
            s = g % S
            pltpu.make_async_copy(eb.at[s], out_h.at[pl.ds(r0, C), pl.ds(D, DE)], sem_w.at[s]).wait()
            pltpu.make_async_copy(tb.at[s], out_h.at[pl.ds(r0, C), pl.ds(2 * D + DE, DT)], sem_w.at[s]).wait()
            pltpu.make_async_copy(fr.at[s], out_h.at[pl.ds(r0, C), pl.ds(0, D)], sem_w.at[s]).wait()
            pltpu.make_async_copy(tr.at[s], out_h.at[pl.ds(r0, C), pl.ds(D + DE, D)], sem_w.at[s]).wait()

        def live(g):
            return jnp.logical_and(g >= 0, (g * NW + wid) < CH)

        def body(g, carry):
            @pl.when(live(g - 2))
            def _():
                stage_drain(g - 2)

            @pl.when(live(g))
            def _():
                stage_in(g)

            @pl.when(live(g - 1))
            def _():
                stage_write_et(g - 1)
                stage_write_ft(g - 1)

            @pl.when(live(g))
            def _():
                stage_gather(g)

            return carry

        lax.fori_loop(0, n_iter + S, body, 0)

    return k(table, fidx, tidx, edge2, time2)


def kernel(update_node_emb, edge_emb, from_idx, to_idx, t, time_w, time_b):
    N, D = update_node_emb.shape
    E, DE = edge_emb.shape
    DT = time_w.shape[0]
    out_w = D + DE + D + DT

    time2 = _time_tc(t, time_w, time_b)
    return _sc_assemble(update_node_emb, from_idx.astype(jnp.int32),
                        to_idx.astype(jnp.int32), edge_emb, time2, out_w)


# speedup vs baseline: 7.2129x; 1.4975x over previous
import functools

import jax
import jax.numpy as jnp
from jax import lax
from jax.experimental import pallas as pl
from jax.experimental.pallas import tpu as pltpu
from jax.experimental.pallas import tpu_sc as plsc


_INV2PI = 0.15915493667125702
_TWOPI = 6.283185307179586
_CC = (0.999999443678766, -0.49999558165578417, 0.04166103279005172,
       -0.001386274731578642, 2.425319249599542e-05, -2.2193949944101022e-07)


def _sc_gather(table, fidx, tidx):
    E = fidx.shape[0]
    N, D = table.shape
    C = 128
    CH = E // C
    info = plsc.get_sparse_core_info()
    NC = info.num_cores
    NW = NC * info.num_subcores
    n_iter = -(-CH // NW)
    mesh = plsc.VectorSubcoreMesh(core_axis_name="c", subcore_axis_name="s")

    S = 2

    @functools.partial(
        pl.kernel,
        mesh=mesh,
        out_type=(jax.ShapeDtypeStruct((E, D), jnp.float32),
                  jax.ShapeDtypeStruct((E, D), jnp.float32)),
        scratch_types=[
            pltpu.VMEM((S * C,), jnp.int32),
            pltpu.VMEM((S * C,), jnp.int32),
            pltpu.VMEM((S * C, D), jnp.float32),
            pltpu.VMEM((S * C, D), jnp.float32),
            pltpu.SemaphoreType.DMA((S,)),
            pltpu.SemaphoreType.DMA((S,)),
            pltpu.SemaphoreType.DMA((S,)),
        ],
    )
    def k(table_h, fidx_h, tidx_h, fout_h, tout_h,
          fiv, tiv, gb, hb, sem_i, sem_g, sem_w):
        wid = lax.axis_index("s") * NC + lax.axis_index("c")

        def stage_in(g):
            r0 = (g * NW + wid) * C
            s = g % S
            pltpu.async_copy(fidx_h.at[pl.ds(r0, C)], fiv.at[pl.ds(s * C, C)], sem_i.at[s])
            pltpu.async_copy(tidx_h.at[pl.ds(r0, C)], tiv.at[pl.ds(s * C, C)], sem_i.at[s])

        def stage_gather(g):
            r0 = (g * NW + wid) * C
            s = g % S
            pltpu.make_async_copy(fidx_h.at[pl.ds(r0, C)], fiv.at[pl.ds(s * C, C)], sem_i.at[s]).wait()
            pltpu.make_async_copy(tidx_h.at[pl.ds(r0, C)], tiv.at[pl.ds(s * C, C)], sem_i.at[s]).wait()
            pltpu.async_copy(table_h.at[fiv.at[pl.ds(s * C, C)]], gb.at[pl.ds(s * C, C)], sem_g.at[s])
            pltpu.async_copy(table_h.at[tiv.at[pl.ds(s * C, C)]], hb.at[pl.ds(s * C, C)], sem_g.at[s])

        def stage_out(g):
            r0 = (g * NW + wid) * C
            s = g % S
            pltpu.make_async_copy(table_h.at[fiv.at[pl.ds(s * C, C)]], gb.at[pl.ds(s * C, C)], sem_g.at[s]).wait()
            pltpu.make_async_copy(table_h.at[tiv.at[pl.ds(s * C, C)]], hb.at[pl.ds(s * C, C)], sem_g.at[s]).wait()
            pltpu.async_copy(gb.at[pl.ds(s * C, C)], fout_h.at[pl.ds(r0, C)], sem_w.at[s])
            pltpu.async_copy(hb.at[pl.ds(s * C, C)], tout_h.at[pl.ds(r0, C)], sem_w.at[s])

        def stage_drain(g):
            r0 = (g * NW + wid) * C
            s = g % S
            pltpu.make_async_copy(gb.at[pl.ds(s * C, C)], fout_h.at[pl.ds(r0, C)], sem_w.at[s]).wait()
            pltpu.make_async_copy(hb.at[pl.ds(s * C, C)], tout_h.at[pl.ds(r0, C)], sem_w.at[s]).wait()

        def live(g):
            return jnp.logical_and(g >= 0, (g * NW + wid) < CH)

        def body(g, carry):
            @pl.when(live(g - 2))
            def _():
                stage_drain(g - 2)

            @pl.when(live(g))
            def _():
                stage_in(g)

            @pl.when(live(g - 1))
            def _():
                stage_out(g - 1)

            @pl.when(live(g))
            def _():
                stage_gather(g)

            return carry

        lax.fori_loop(0, n_iter + S, body, 0)

    return k(table, fidx, tidx)


def _assemble_tc(from2, to2, edge2, t, time_w, time_b, out_w):
    E, D = from2.shape
    DE = edge2.shape[-1]
    DT = time_w.shape[0]
    BR = 20
    C = 128
    B = BR * C
    G = E // B
    assert E == G * B

    def body(f_ref, g_ref, e_ref, t_ref, w_ref, b_ref, o_ref):
        x = t_ref[0][:, :, None] * w_ref[...] + b_ref[...]
        r = x - jnp.floor(x * _INV2PI + 0.5) * _TWOPI
        u = r * r
        p = jnp.float32(_CC[5])
        for c in (_CC[4], _CC[3], _CC[2], _CC[1], _CC[0]):
            p = p * u + c
        o_ref[...] = jnp.concatenate(
            [f_ref[...], e_ref[...], g_ref[...], p.reshape(B, DT)], axis=-1)

    return pl.pallas_call(
        body,
        grid=(G,),
        in_specs=[
            pl.BlockSpec((B, D), lambda i: (i, 0)),
            pl.BlockSpec((B, D), lambda i: (i, 0)),
            pl.BlockSpec((B, DE), lambda i: (i, 0)),
            pl.BlockSpec((1, BR, C), lambda i: (i, 0, 0)),
            pl.BlockSpec((1, 1, DT), lambda i: (0, 0, 0)),
            pl.BlockSpec((1, 1, DT), lambda i: (0, 0, 0)),
        ],
        out_specs=pl.BlockSpec((B, out_w), lambda i: (i, 0)),
        out_shape=jax.ShapeDtypeStruct((E, out_w), jnp.float32),
    )(from2, to2, edge2, t.reshape(G, BR, C),
      time_w.reshape(1, 1, DT), time_b.reshape(1, 1, DT))


def kernel(update_node_emb, edge_emb, from_idx, to_idx, t, time_w, time_b):
    N, D = update_node_emb.shape
    E, DE = edge_emb.shape
    DT = time_w.shape[0]
    out_w = D + DE + D + DT

    from2, to2 = _sc_gather(update_node_emb, from_idx.astype(jnp.int32),
                            to_idx.astype(jnp.int32))
    return _assemble_tc(from2, to2, edge_emb, t, time_w, time_b, out_w)


# speedup vs baseline: 15.6117x; 2.1644x over previous
import functools

import jax
import jax.numpy as jnp
from jax import lax
from jax.experimental import pallas as pl
from jax.experimental.pallas import tpu as pltpu
from jax.experimental.pallas import tpu_sc as plsc


_INV2PI = 0.15915493667125702
_TWOPI = 6.283185307179586
_CC = (0.999999443678766, -0.49999558165578417, 0.04166103279005172,
       -0.001386274731578642, 2.425319249599542e-05, -2.2193949944101022e-07)


def _sc_gather(table, fidx, tidx):
    E = fidx.shape[0]
    N, D = table.shape
    C = 128
    CH = E // C
    info = plsc.get_sparse_core_info()
    NC = info.num_cores
    NW = NC * info.num_subcores
    n_iter = -(-CH // NW)
    mesh = plsc.VectorSubcoreMesh(core_axis_name="c", subcore_axis_name="s")

    S = 2

    @functools.partial(
        pl.kernel,
        mesh=mesh,
        out_type=(jax.ShapeDtypeStruct((E, D), jnp.float32),
                  jax.ShapeDtypeStruct((E, D), jnp.float32)),
        scratch_types=[
            pltpu.VMEM((S * C,), jnp.int32),
            pltpu.VMEM((S * C,), jnp.int32),
            pltpu.VMEM((S * C, D), jnp.float32),
            pltpu.VMEM((S * C, D), jnp.float32),
            pltpu.SemaphoreType.DMA((S,)),
            pltpu.SemaphoreType.DMA((S,)),
            pltpu.SemaphoreType.DMA((S,)),
        ],
    )
    def k(table_h, fidx_h, tidx_h, fout_h, tout_h,
          fiv, tiv, gb, hb, sem_i, sem_g, sem_w):
        wid = lax.axis_index("s") * NC + lax.axis_index("c")

        def stage_in(g):
            r0 = (g * NW + wid) * C
            s = g % S
            pltpu.async_copy(fidx_h.at[pl.ds(r0, C)], fiv.at[pl.ds(s * C, C)], sem_i.at[s])
            pltpu.async_copy(tidx_h.at[pl.ds(r0, C)], tiv.at[pl.ds(s * C, C)], sem_i.at[s])

        def stage_gather(g):
            r0 = (g * NW + wid) * C
            s = g % S
            pltpu.make_async_copy(fidx_h.at[pl.ds(r0, C)], fiv.at[pl.ds(s * C, C)], sem_i.at[s]).wait()
            pltpu.make_async_copy(tidx_h.at[pl.ds(r0, C)], tiv.at[pl.ds(s * C, C)], sem_i.at[s]).wait()
            pltpu.async_copy(table_h.at[fiv.at[pl.ds(s * C, C)]], gb.at[pl.ds(s * C, C)], sem_g.at[s])
            pltpu.async_copy(table_h.at[tiv.at[pl.ds(s * C, C)]], hb.at[pl.ds(s * C, C)], sem_g.at[s])

        def stage_out(g):
            r0 = (g * NW + wid) * C
            s = g % S
            pltpu.make_async_copy(table_h.at[fiv.at[pl.ds(s * C, C)]], gb.at[pl.ds(s * C, C)], sem_g.at[s]).wait()
            pltpu.make_async_copy(table_h.at[tiv.at[pl.ds(s * C, C)]], hb.at[pl.ds(s * C, C)], sem_g.at[s]).wait()
            pltpu.async_copy(gb.at[pl.ds(s * C, C)], fout_h.at[pl.ds(r0, C)], sem_w.at[s])
            pltpu.async_copy(hb.at[pl.ds(s * C, C)], tout_h.at[pl.ds(r0, C)], sem_w.at[s])

        def stage_drain(g):
            r0 = (g * NW + wid) * C
            s = g % S
            pltpu.make_async_copy(gb.at[pl.ds(s * C, C)], fout_h.at[pl.ds(r0, C)], sem_w.at[s]).wait()
            pltpu.make_async_copy(hb.at[pl.ds(s * C, C)], tout_h.at[pl.ds(r0, C)], sem_w.at[s]).wait()

        def live(g):
            return jnp.logical_and(g >= 0, (g * NW + wid) < CH)

        def body(g, carry):
            @pl.when(live(g - 2))
            def _():
                stage_drain(g - 2)

            @pl.when(live(g))
            def _():
                stage_in(g)

            @pl.when(live(g - 1))
            def _():
                stage_out(g - 1)

            @pl.when(live(g))
            def _():
                stage_gather(g)

            return carry

        lax.fori_loop(0, n_iter + S, body, 0)

    return k(table, fidx, tidx)


def _assemble_tc(from2, to2, edge2, t, time_w, time_b, out_w):
    E, D = from2.shape
    DE = edge2.shape[-1]
    DT = time_w.shape[0]
    B = 2560
    G = E // B
    assert E == G * B

    def body(f_ref, g_ref, e_ref, t_ref, w_ref, b_ref, o_ref):
        x = w_ref[...] * t_ref[...] + b_ref[...]
        r = x - jnp.floor(x * _INV2PI + 0.5) * _TWOPI
        u = r * r
        p = jnp.float32(_CC[5])
        for c in (_CC[4], _CC[3], _CC[2], _CC[1], _CC[0]):
            p = p * u + c
        o_ref[...] = jnp.concatenate(
            [f_ref[...].T, e_ref[...], g_ref[...].T, p], axis=0)

    out_t = pl.pallas_call(
        body,
        grid=(G,),
        in_specs=[
            pl.BlockSpec((B, D), lambda i: (i, 0)),
            pl.BlockSpec((B, D), lambda i: (i, 0)),
            pl.BlockSpec((DE, B), lambda i: (0, i)),
            pl.BlockSpec((1, B), lambda i: (0, i)),
            pl.BlockSpec((DT, 1), lambda i: (0, 0)),
            pl.BlockSpec((DT, 1), lambda i: (0, 0)),
        ],
        out_specs=pl.BlockSpec((out_w, B), lambda i: (0, i)),
        out_shape=jax.ShapeDtypeStruct((out_w, E), jnp.float32),
    )(from2, to2, edge2.T, t.reshape(1, E),
      time_w.reshape(DT, 1), time_b.reshape(DT, 1))
    return out_t.T


def kernel(update_node_emb, edge_emb, from_idx, to_idx, t, time_w, time_b):
    N, D = update_node_emb.shape
    E, DE = edge_emb.shape
    DT = time_w.shape[0]
    out_w = D + DE + D + DT

    from2, to2 = _sc_gather(update_node_emb, from_idx.astype(jnp.int32),
                            to_idx.astype(jnp.int32))
    return _assemble_tc(from2, to2, edge_emb, t, time_w, time_b, out_w)
